# Initial kernel scaffold; baseline (speedup 1.0000x reference)
import jax
import jax.numpy as jnp
from jax.experimental import pallas as pl


def kernel(tokens, table):
    raise NotImplementedError("write your pallas kernel here")


# speedup vs baseline: 5.0985x; 5.0985x over previous
import functools

import jax
import jax.numpy as jnp
from jax import lax
from jax.experimental import pallas as pl
from jax.experimental.pallas import tpu as pltpu
from jax.experimental.pallas import tpu_sc as plsc

V = 100000
D = 128
BOS = 1
B = 4096
L = 200
N = B * L
NC, NS = 2, 16
NW = NC * NS
PER_W = N // NW
C = 128
CHUNKS_W = PER_W // C
NBUF = 4
IDX_ROWS = N // C
ROWS_BLK = 4000
TOK_BLK = IDX_ROWS // (V // ROWS_BLK)


def _prep_body(head_ref, tab_ref, tok_ref, t2_ref, idx_ref):
    t2_ref[...] = tab_ref[...] + head_ref[BOS, :][None, :]
    i = pl.program_id(0)
    r = lax.broadcasted_iota(jnp.int32, (TOK_BLK, C), 0)
    c = lax.broadcasted_iota(jnp.int32, (TOK_BLK, C), 1)
    n = (i * TOK_BLK + r) * C + c
    idx_ref[...] = jnp.where(n % L == 0, BOS, tok_ref[...])


def _prep(table, tok2d):
    return pl.pallas_call(
        _prep_body,
        grid=(V // ROWS_BLK,),
        in_specs=[
            pl.BlockSpec((8, D), lambda i: (0, 0)),
            pl.BlockSpec((ROWS_BLK, D), lambda i: (i, 0)),
            pl.BlockSpec((TOK_BLK, C), lambda i: (i, 0)),
        ],
        out_specs=[
            pl.BlockSpec((ROWS_BLK, D), lambda i: (i, 0)),
            pl.BlockSpec((TOK_BLK, C), lambda i: (i, 0)),
        ],
        out_shape=[
            jax.ShapeDtypeStruct((V, D), jnp.float32),
            jax.ShapeDtypeStruct((IDX_ROWS, C), jnp.int32),
        ],
    )(table, table, tok2d)


def _sc_gather_body(table2_hbm, idx_hbm, out_hbm, idx_v, *rest):
    bufs = rest[:NBUF]
    gsems = rest[NBUF:2 * NBUF]
    ssems = rest[2 * NBUF:3 * NBUF]
    wid = lax.axis_index("s") * NC + lax.axis_index("c")
    chunk0 = wid * CHUNKS_W
    row0 = wid * PER_W
    pltpu.sync_copy(idx_hbm.at[pl.ds(chunk0, CHUNKS_W)], idx_v)

    def round_body(g, carry):
        gh = []
        for b in range(NBUF):
            j = g * NBUF + b
            gh.append(pltpu.async_copy(
                table2_hbm.at[idx_v.at[j]], bufs[b], gsems[b]))
        sh = []
        for b in range(NBUF):
            j = g * NBUF + b
            gh[b].wait()
            sh.append(pltpu.async_copy(
                bufs[b], out_hbm.at[pl.ds(row0 + j * C, C)], ssems[b]))
        for h in sh:
            h.wait()
        return carry

    lax.fori_loop(0, CHUNKS_W // NBUF, round_body, 0)


@functools.lru_cache(maxsize=1)
def _sc_gather():
    mesh = plsc.VectorSubcoreMesh(
        core_axis_name="c", subcore_axis_name="s",
        num_cores=NC, num_subcores=NS)
    return pl.kernel(
        _sc_gather_body,
        out_type=jax.ShapeDtypeStruct((N, D), jnp.float32),
        mesh=mesh,
        scratch_types=[
            pltpu.VMEM((CHUNKS_W, C), jnp.int32),
            *[pltpu.VMEM((C, D), jnp.float32) for _ in range(NBUF)],
            *[pltpu.SemaphoreType.DMA for _ in range(2 * NBUF)],
        ],
    )


def kernel(tokens, table):
    tok2d = tokens.astype(jnp.int32).reshape(IDX_ROWS, C)
    table2, idx = _prep(table, tok2d)
    out = _sc_gather()(table2, idx)
    return out.reshape(B, L, D)


# speedup vs baseline: 5.1000x; 1.0003x over previous
import functools

import jax
import jax.numpy as jnp
from jax import lax
from jax.experimental import pallas as pl
from jax.experimental.pallas import tpu as pltpu
from jax.experimental.pallas import tpu_sc as plsc

V = 100000
D = 128
BOS = 1
B = 4096
L = 200
N = B * L
NC, NS = 2, 16
NW = NC * NS
PER_W = N // NW
C = 128
CHUNKS_W = PER_W // C
NBUF = 5
IDX_ROWS = N // C
ROWS_BLK = 4000
TOK_BLK = IDX_ROWS // (V // ROWS_BLK)


def _prep_body(head_ref, tab_ref, tok_ref, t2_ref, idx_ref):
    t2_ref[...] = tab_ref[...] + head_ref[BOS, :][None, :]
    i = pl.program_id(0)
    r = lax.broadcasted_iota(jnp.int32, (TOK_BLK, C), 0)
    c = lax.broadcasted_iota(jnp.int32, (TOK_BLK, C), 1)
    n = (i * TOK_BLK + r) * C + c
    idx_ref[...] = jnp.where(n % L == 0, BOS, tok_ref[...])


def _prep(table, tok2d):
    return pl.pallas_call(
        _prep_body,
        grid=(V // ROWS_BLK,),
        in_specs=[
            pl.BlockSpec((8, D), lambda i: (0, 0)),
            pl.BlockSpec((ROWS_BLK, D), lambda i: (i, 0)),
            pl.BlockSpec((TOK_BLK, C), lambda i: (i, 0)),
        ],
        out_specs=[
            pl.BlockSpec((ROWS_BLK, D), lambda i: (i, 0)),
            pl.BlockSpec((TOK_BLK, C), lambda i: (i, 0)),
        ],
        out_shape=[
            jax.ShapeDtypeStruct((V, D), jnp.float32),
            jax.ShapeDtypeStruct((IDX_ROWS, C), jnp.int32),
        ],
    )(table, table, tok2d)


def _sc_gather_body(table2_hbm, idx_hbm, out_hbm, idx_v, *rest):
    bufs = rest[:NBUF]
    gsems = rest[NBUF:2 * NBUF]
    ssems = rest[2 * NBUF:3 * NBUF]
    wid = lax.axis_index("s") * NC + lax.axis_index("c")
    chunk0 = wid * CHUNKS_W
    row0 = wid * PER_W
    pltpu.sync_copy(idx_hbm.at[pl.ds(chunk0, CHUNKS_W)], idx_v)

    def round_body(g, carry):
        j0 = g * NBUF
        gh = []
        for b in range(NBUF):
            @pl.when(g > 0)
            def _drain_prev_store(b=b):
                pltpu.make_async_copy(
                    bufs[b], out_hbm.at[pl.ds(row0, C)], ssems[b]).wait()
            gh.append(pltpu.async_copy(
                table2_hbm.at[idx_v.at[j0 + b]], bufs[b], gsems[b]))
        for b in range(NBUF):
            gh[b].wait()
            pltpu.async_copy(
                bufs[b], out_hbm.at[pl.ds(row0 + (j0 + b) * C, C)], ssems[b])
        return carry

    lax.fori_loop(0, CHUNKS_W // NBUF, round_body, 0)
    for b in range(NBUF):
        pltpu.make_async_copy(
            bufs[b], out_hbm.at[pl.ds(row0, C)], ssems[b]).wait()


@functools.lru_cache(maxsize=1)
def _sc_gather():
    mesh = plsc.VectorSubcoreMesh(
        core_axis_name="c", subcore_axis_name="s",
        num_cores=NC, num_subcores=NS)
    return pl.kernel(
        _sc_gather_body,
        out_type=jax.ShapeDtypeStruct((N, D), jnp.float32),
        mesh=mesh,
        scratch_types=[
            pltpu.VMEM((CHUNKS_W, C), jnp.int32),
            *[pltpu.VMEM((C, D), jnp.float32) for _ in range(NBUF)],
            *[pltpu.SemaphoreType.DMA for _ in range(2 * NBUF)],
        ],
    )


def kernel(tokens, table):
    tok2d = tokens.astype(jnp.int32).reshape(IDX_ROWS, C)
    table2, idx = _prep(table, tok2d)
    out = _sc_gather()(table2, idx)
    return out.reshape(B, L, D)


# speedup vs baseline: 5.4969x; 1.0778x over previous
import functools

import jax
import jax.numpy as jnp
from jax import lax
from jax.experimental import pallas as pl
from jax.experimental.pallas import tpu as pltpu
from jax.experimental.pallas import tpu_sc as plsc

V = 100000
D = 128
BOS = 1
B = 4096
L = 200
N = B * L
NC, NS = 2, 16
NW = NC * NS
PER_W = N // NW
C = 128
CHUNKS_W = PER_W // C
NBUF = 5
NLANE = 16
NV = D // NLANE


def _sc_body(table_hbm, idx_hbm, out_hbm, idx_v, bos_v, *rest):
    bufs = rest[:NBUF]
    gsems = rest[NBUF:2 * NBUF]
    ssems = rest[2 * NBUF:3 * NBUF]
    wid = lax.axis_index("s") * NC + lax.axis_index("c")
    row0 = wid * PER_W

    pltpu.sync_copy(idx_hbm.at[pl.ds(row0, PER_W)], idx_v)
    pltpu.sync_copy(table_hbm.at[pl.ds(BOS, 1)], bos_v)

    lane = lax.iota(jnp.int32, NLANE)
    for m in range(PER_W // L):
        off = m * L
        sl = pl.ds((off // NLANE) * NLANE, NLANE)
        idx_v[sl] = jnp.where(lane == off % NLANE, BOS, idx_v[sl])

    bosv = [bos_v[0, pl.ds(v * NLANE, NLANE)] for v in range(NV)]

    NR = CHUNKS_W // NBUF
    for b in range(NBUF):
        pltpu.async_copy(
            table_hbm.at[idx_v.at[pl.ds(b * C, C)]], bufs[b], gsems[b])

    def round_body(g, carry):
        j0 = g * NBUF
        for b in range(NBUF):
            pltpu.make_async_copy(
                table_hbm.at[idx_v.at[pl.ds((j0 + b) * C, C)]],
                bufs[b], gsems[b]).wait()

            @pl.when(g > 0)
            def _drain_prev_store(b=b):
                pltpu.make_async_copy(
                    bufs[b], out_hbm.at[pl.ds(row0, C)], ssems[b]).wait()

            buf = bufs[b]

            @plsc.parallel_loop(0, C, 1, unroll=2)
            def row_body(r):
                for v in range(NV):
                    sl = pl.ds(v * NLANE, NLANE)
                    buf[r, sl] = buf[r, sl] + bosv[v]

            pltpu.async_copy(
                buf, out_hbm.at[pl.ds(row0 + (j0 + b) * C, C)], ssems[b])

            @pl.when(g < NR - 1)
            def _next_gather(b=b, j0=j0):
                pltpu.async_copy(
                    table_hbm.at[idx_v.at[pl.ds((j0 + NBUF + b) * C, C)]],
                    bufs[b], gsems[b])
        return carry

    lax.fori_loop(0, NR, round_body, 0)
    for b in range(NBUF):
        pltpu.make_async_copy(
            bufs[b], out_hbm.at[pl.ds(row0, C)], ssems[b]).wait()


@functools.lru_cache(maxsize=1)
def _sc_kernel():
    mesh = plsc.VectorSubcoreMesh(
        core_axis_name="c", subcore_axis_name="s",
        num_cores=NC, num_subcores=NS)
    return pl.kernel(
        _sc_body,
        out_type=jax.ShapeDtypeStruct((N, D), jnp.float32),
        mesh=mesh,
        scratch_types=[
            pltpu.VMEM((PER_W,), jnp.int32),
            pltpu.VMEM((1, D), jnp.float32),
            *[pltpu.VMEM((C, D), jnp.float32) for _ in range(NBUF)],
            *[pltpu.SemaphoreType.DMA for _ in range(2 * NBUF)],
        ],
    )


def kernel(tokens, table):
    idx = tokens.astype(jnp.int32).reshape(N)
    out = _sc_kernel()(table, idx)
    return out.reshape(B, L, D)
